# baseline (speedup 1.0000x reference)
import functools

import jax
import jax.numpy as jnp
from jax import lax
from jax.experimental import pallas as pl
from jax.experimental.pallas import tpu as pltpu
from jax.experimental.pallas import tpu_sc as plsc

NUM_EMBEDDINGS = 1000000
EMBEDDING_DIM = 32
BATCH = 16384
N_FIELDS = 26

TOTAL = BATCH * N_FIELDS
NUM_CORES = 2
NUM_SUBCORES = 16
NW = NUM_CORES * NUM_SUBCORES
L = 16

W4 = NUM_EMBEDDINGS // 4
IDX_COLS = 128
IDX_ROWS = TOTAL // IDX_COLS
ROWS_PER_W = IDX_ROWS // NW
CHUNK = 128
N_CHUNKS = ROWS_PER_W
OUT_COLS = 128
OUT_ROWS = TOTAL * EMBEDDING_DIM // OUT_COLS
OUT_R_PER_CHUNK = CHUNK * EMBEDDING_DIM // OUT_COLS
NBUF = 4


@functools.partial(
    pl.kernel,
    out_type=jax.ShapeDtypeStruct((OUT_ROWS, OUT_COLS), jnp.float32),
    mesh=plsc.VectorSubcoreMesh(core_axis_name="c", subcore_axis_name="s"),
    scratch_types=[
        pltpu.VMEM((ROWS_PER_W, IDX_COLS), jnp.int32),
        pltpu.VMEM((ROWS_PER_W, IDX_COLS), jnp.int32),
        pltpu.VMEM((ROWS_PER_W, IDX_COLS), jnp.int32),
        pltpu.VMEM((NBUF, CHUNK, 128), jnp.float32),
        pltpu.VMEM((2, OUT_R_PER_CHUNK, OUT_COLS), jnp.float32),
        pltpu.SemaphoreType.DMA,
        pltpu.SemaphoreType.DMA,
    ],
    compiler_params=pltpu.CompilerParams(
        use_tc_tiling_on_sc=True, needs_layout_passes=False),
)
def _gather_sc(table4, idx_hbm, out_hbm, idx_v, offs_v, sub_v, rows4_v,
               outb_v, sem_g, sem_s):
    wid = lax.axis_index("s") * NUM_CORES + lax.axis_index("c")
    base = wid * ROWS_PER_W

    pltpu.sync_copy(idx_hbm.at[pl.ds(base, ROWS_PER_W)], idx_v)

    def prep_row(r, _):
        for g in range(IDX_COLS // L):
            v = idx_v[r, pl.ds(g * L, L)]
            offs_v[r, pl.ds(g * L, L)] = lax.shift_right_logical(v, 2)
            sub_v[r, pl.ds(g * L, L)] = lax.shift_left(
                lax.bitwise_and(v, 3), 5)
        return 0

    lax.fori_loop(0, ROWS_PER_W, prep_row, 0)

    def issue_gather(i):
        pltpu.async_copy(
            table4.at[offs_v.at[i]], rows4_v.at[lax.rem(i, NBUF)], sem_g)

    def wait_gather():
        pltpu.make_async_copy(
            table4.at[offs_v.at[0]], rows4_v.at[0], sem_g).wait()

    def wait_store():
        pltpu.make_async_copy(
            outb_v.at[0], out_hbm.at[pl.ds(0, OUT_R_PER_CHUNK)], sem_s).wait()

    lanes = lax.iota(jnp.int32, L)

    def select(i, b):
        ob = lax.rem(i, 2)
        for g in range(CHUNK // L):
            krel = lanes + g * L
            subs = sub_v[i, pl.ds(g * L, L)]
            dst_flat = krel * EMBEDDING_DIM
            for c in range(EMBEDDING_DIM):
                vals = plsc.load_gather(rows4_v.at[b], [krel, subs + c])
                f = dst_flat + c
                plsc.store_scatter(
                    outb_v.at[ob],
                    [lax.shift_right_logical(f, 7),
                     lax.bitwise_and(f, OUT_COLS - 1)], vals)

    issue_gather(0)
    issue_gather(1)
    issue_gather(2)

    def body(i, _):
        b = lax.rem(i, NBUF)
        wait_gather()

        @pl.when(i + 3 < N_CHUNKS)
        def _():
            issue_gather(i + 3)

        @pl.when(i >= 2)
        def _():
            wait_store()

        select(i, b)
        pltpu.async_copy(
            outb_v.at[lax.rem(i, 2)],
            out_hbm.at[pl.ds((base + i) * OUT_R_PER_CHUNK, OUT_R_PER_CHUNK)],
            sem_s)
        return 0

    lax.fori_loop(0, N_CHUNKS, body, 0)
    wait_store()
    wait_store()


def kernel(input, weight):
    table4 = weight.reshape(W4, 128)
    idx2 = input.reshape(IDX_ROWS, IDX_COLS)
    out = _gather_sc(table4, idx2)
    return out.reshape(BATCH, N_FIELDS, EMBEDDING_DIM)


# speedup vs baseline: 1.1274x; 1.1274x over previous
import functools

import jax
import jax.numpy as jnp
from jax import lax
from jax.experimental import pallas as pl
from jax.experimental.pallas import tpu as pltpu
from jax.experimental.pallas import tpu_sc as plsc

NUM_EMBEDDINGS = 1000000
EMBEDDING_DIM = 32
BATCH = 16384
N_FIELDS = 26

TOTAL = BATCH * N_FIELDS
NUM_CORES = 2
NUM_SUBCORES = 16
NW = NUM_CORES * NUM_SUBCORES
L = 16

W4 = NUM_EMBEDDINGS // 4
IDX_COLS = 128
IDX_ROWS = TOTAL // IDX_COLS
ROWS_PER_W = IDX_ROWS // NW
CHUNK = 128
N_CHUNKS = ROWS_PER_W
OUT_COLS = 128
OUT_ROWS = TOTAL * EMBEDDING_DIM // OUT_COLS
OUT_R_PER_CHUNK = CHUNK * EMBEDDING_DIM // OUT_COLS
NBUF = 4


@functools.partial(
    pl.kernel,
    out_type=jax.ShapeDtypeStruct((OUT_ROWS, OUT_COLS), jnp.float32),
    mesh=plsc.VectorSubcoreMesh(core_axis_name="c", subcore_axis_name="s"),
    scratch_types=[
        pltpu.VMEM((ROWS_PER_W, IDX_COLS), jnp.int32),
        pltpu.VMEM((ROWS_PER_W, IDX_COLS), jnp.int32),
        pltpu.VMEM((ROWS_PER_W, IDX_COLS), jnp.int32),
        pltpu.VMEM((NBUF, CHUNK, 128), jnp.float32),
        pltpu.VMEM((2, OUT_R_PER_CHUNK, OUT_COLS), jnp.float32),
        pltpu.SemaphoreType.DMA,
        pltpu.SemaphoreType.DMA,
    ],
    compiler_params=pltpu.CompilerParams(
        use_tc_tiling_on_sc=True, needs_layout_passes=False),
)
def _gather_sc(table4, idx_hbm, out_hbm, idx_v, offs_v, sub_v, rows4_v,
               outb_v, sem_g, sem_s):
    wid = lax.axis_index("s") * NUM_CORES + lax.axis_index("c")
    base = wid * ROWS_PER_W

    pltpu.sync_copy(idx_hbm.at[pl.ds(base, ROWS_PER_W)], idx_v)

    def prep_row(r, _):
        for g in range(IDX_COLS // L):
            v = idx_v[r, pl.ds(g * L, L)]
            offs_v[r, pl.ds(g * L, L)] = lax.shift_right_logical(v, 2)
            sub_v[r, pl.ds(g * L, L)] = lax.shift_left(
                lax.bitwise_and(v, 3), 5)
        return 0

    lax.fori_loop(0, ROWS_PER_W, prep_row, 0)

    def issue_gather(i):
        pltpu.async_copy(
            table4.at[offs_v.at[i]], rows4_v.at[lax.rem(i, NBUF)], sem_g)

    def wait_gather():
        pltpu.make_async_copy(
            table4.at[offs_v.at[0]], rows4_v.at[0], sem_g).wait()

    def wait_store():
        pltpu.make_async_copy(
            outb_v.at[0], out_hbm.at[pl.ds(0, OUT_R_PER_CHUNK)], sem_s).wait()

    lanes = lax.iota(jnp.int32, L)

    rowbase = lax.shift_right_logical(lanes, 2)
    colbase = lax.shift_left(lax.bitwise_and(lanes, 3), 5)

    def select(i, b):
        ob = lax.rem(i, 2)
        for g in range(CHUNK // L):
            krel = lanes + g * L
            subs = sub_v[i, pl.ds(g * L, L)]
            row_vec = rowbase + (4 * g)
            vals = [
                plsc.load_gather(rows4_v.at[b], [krel, subs + c])
                for c in range(EMBEDDING_DIM)
            ]
            for c in range(EMBEDDING_DIM):
                plsc.store_scatter(
                    outb_v.at[ob], [row_vec, colbase + c], vals[c])

    issue_gather(0)
    issue_gather(1)
    issue_gather(2)

    def body(i, _):
        b = lax.rem(i, NBUF)
        wait_gather()

        @pl.when(i + 3 < N_CHUNKS)
        def _():
            issue_gather(i + 3)

        @pl.when(i >= 2)
        def _():
            wait_store()

        select(i, b)
        pltpu.async_copy(
            outb_v.at[lax.rem(i, 2)],
            out_hbm.at[pl.ds((base + i) * OUT_R_PER_CHUNK, OUT_R_PER_CHUNK)],
            sem_s)
        return 0

    lax.fori_loop(0, N_CHUNKS, body, 0)
    wait_store()
    wait_store()


def kernel(input, weight):
    table4 = weight.reshape(W4, 128)
    idx2 = input.reshape(IDX_ROWS, IDX_COLS)
    out = _gather_sc(table4, idx2)
    return out.reshape(BATCH, N_FIELDS, EMBEDDING_DIM)


# speedup vs baseline: 1.5686x; 1.3914x over previous
import functools

import jax
import jax.numpy as jnp
from jax import lax
from jax.experimental import pallas as pl
from jax.experimental.pallas import tpu as pltpu
from jax.experimental.pallas import tpu_sc as plsc

NUM_EMBEDDINGS = 1000000
EMBEDDING_DIM = 32
BATCH = 16384
N_FIELDS = 26

TOTAL = BATCH * N_FIELDS
NUM_CORES = 2
NUM_SUBCORES = 16
NW = NUM_CORES * NUM_SUBCORES
L = 16

HALF = EMBEDDING_DIM // 2
W2 = NUM_EMBEDDINGS * 2
B_PER_W = TOTAL // NW
CHUNK = 832
N_CHUNKS = B_PER_W // CHUNK
NBUF = 3


@functools.partial(
    pl.kernel,
    out_type=jax.ShapeDtypeStruct((TOTAL * 2, HALF), jnp.float32),
    mesh=plsc.VectorSubcoreMesh(core_axis_name="c", subcore_axis_name="s"),
    scratch_types=[
        pltpu.VMEM((B_PER_W,), jnp.int32),
        pltpu.VMEM((2 * B_PER_W,), jnp.int32),
        pltpu.VMEM((NBUF, 2 * CHUNK, HALF), jnp.float32),
        pltpu.SemaphoreType.DMA,
        pltpu.SemaphoreType.DMA,
    ],
    compiler_params=pltpu.CompilerParams(
        use_tc_tiling_on_sc=False, needs_layout_passes=False),
)
def _gather_sc(table2, idx_hbm, out_hbm, idx_v, offs_v, rows_v, sem_g, sem_s):
    wid = lax.axis_index("s") * NUM_CORES + lax.axis_index("c")
    base = wid * B_PER_W

    pltpu.sync_copy(idx_hbm.at[pl.ds(base, B_PER_W)], idx_v)

    lanes = lax.iota(jnp.int32, L)

    def prep(q, _):
        v = idx_v[pl.ds(q * L, L)]
        v2 = lax.shift_left(v, 1)
        pos = 2 * q * L + lax.shift_left(lanes, 1)
        plsc.store_scatter(offs_v, [pos], v2)
        plsc.store_scatter(offs_v, [pos + 1], v2 + 1)
        return 0

    lax.fori_loop(0, B_PER_W // L, prep, 0)

    def issue_gather(i):
        pltpu.async_copy(
            table2.at[offs_v.at[pl.ds(i * 2 * CHUNK, 2 * CHUNK)]],
            rows_v.at[lax.rem(i, NBUF)], sem_g)

    def wait_gather():
        pltpu.make_async_copy(
            table2.at[offs_v.at[pl.ds(0, 2 * CHUNK)]], rows_v.at[0],
            sem_g).wait()

    def wait_store():
        pltpu.make_async_copy(
            rows_v.at[0], out_hbm.at[pl.ds(0, 2 * CHUNK)], sem_s).wait()

    issue_gather(0)
    issue_gather(1)

    def body(i, _):
        b = lax.rem(i, NBUF)
        wait_gather()

        @pl.when(i + 2 < N_CHUNKS)
        def _():
            @pl.when(i >= 1)
            def _():
                wait_store()
            issue_gather(i + 2)

        pltpu.async_copy(
            rows_v.at[b],
            out_hbm.at[pl.ds((base + i * CHUNK) * 2, 2 * CHUNK)], sem_s)
        return 0

    lax.fori_loop(0, N_CHUNKS, body, 0)
    for _ in range(3):
        wait_store()


def kernel(input, weight):
    table2 = weight.reshape(W2, HALF)
    idx_f = input.reshape(TOTAL)
    out = _gather_sc(table2, idx_f)
    return out.reshape(BATCH, N_FIELDS, EMBEDDING_DIM)


# speedup vs baseline: 1.5736x; 1.0032x over previous
import functools

import jax
import jax.numpy as jnp
from jax import lax
from jax.experimental import pallas as pl
from jax.experimental.pallas import tpu as pltpu
from jax.experimental.pallas import tpu_sc as plsc

NUM_EMBEDDINGS = 1000000
EMBEDDING_DIM = 32
BATCH = 16384
N_FIELDS = 26

NUM_CORES = 2
NUM_SUBCORES = 16
NW = NUM_CORES * NUM_SUBCORES
ROWS_PER_W = BATCH // NW
R_CHUNK = 32
N_CHUNKS = ROWS_PER_W // R_CHUNK
NBUF = 4


@functools.partial(
    pl.kernel,
    out_type=jax.ShapeDtypeStruct((BATCH, N_FIELDS, EMBEDDING_DIM), jnp.float32),
    mesh=plsc.VectorSubcoreMesh(core_axis_name="c", subcore_axis_name="s"),
    scratch_types=[
        pltpu.VMEM((ROWS_PER_W, N_FIELDS), jnp.int32),
        pltpu.VMEM((NBUF, R_CHUNK, N_FIELDS, EMBEDDING_DIM), jnp.float32),
        pltpu.SemaphoreType.DMA,
        pltpu.SemaphoreType.DMA,
    ],
    compiler_params=pltpu.CompilerParams(use_tc_tiling_on_sc=False),
)
def _gather_sc(table_hbm, idx_hbm, out_hbm, idx_v, rows_v, sem_g, sem_s):
    wid = lax.axis_index("s") * NUM_CORES + lax.axis_index("c")
    base = wid * ROWS_PER_W

    pltpu.sync_copy(idx_hbm.at[pl.ds(base, ROWS_PER_W)], idx_v)

    def gather(i):
        b = i % NBUF

        def issue(j, _):
            pltpu.async_copy(
                table_hbm.at[idx_v.at[i * R_CHUNK + j]],
                rows_v.at[b, j], sem_g)
            return 0

        lax.fori_loop(0, R_CHUNK, issue, 0)
        return pltpu.make_async_copy(
            out_hbm.at[pl.ds(0, R_CHUNK)], rows_v.at[b], sem_g)

    def store(i):
        return pltpu.async_copy(
            rows_v.at[i % NBUF],
            out_hbm.at[pl.ds(base + i * R_CHUNK, R_CHUNK)], sem_s)

    gathers = [gather(0), gather(1)]
    stores = []
    for i in range(N_CHUNKS):
        gathers[i].wait()
        nxt = i + 2
        if nxt < N_CHUNKS:
            if nxt >= NBUF:
                stores[nxt - NBUF].wait()
            gathers.append(gather(nxt))
        stores.append(store(i))
    for j in range(max(0, N_CHUNKS - NBUF), N_CHUNKS):
        stores[j].wait()


def kernel(input, weight):
    return _gather_sc(weight, input)
